# baseline (speedup 1.0000x reference)
import functools

import jax
import jax.numpy as jnp
from jax import lax
from jax.experimental import pallas as pl
from jax.experimental.pallas import tpu as pltpu
from jax.experimental.pallas import tpu_sc as plsc

_BATCH = 4096
_SEQ = 200
_EMBED = 64
_B = _BATCH * _SEQ

_info = plsc.get_sparse_core_info()
_NC, _NS = _info.num_cores, _info.num_subcores
_NW = _NC * _NS
_PER_W = _B // _NW
_NSLOT = 4
_C = 400
_CHUNKS = _PER_W // _C
_ITERS = _CHUNKS // _NSLOT

_mesh = plsc.VectorSubcoreMesh(core_axis_name="c", subcore_axis_name="s")


@functools.partial(
    pl.kernel,
    mesh=_mesh,
    out_type=jax.ShapeDtypeStruct((_B, _EMBED), jnp.float32),
    scratch_types=(
        [pltpu.VMEM((_C,), jnp.int32) for _ in range(_NSLOT)]
        + [pltpu.VMEM((_C, _EMBED), jnp.float32) for _ in range(_NSLOT)]
        + [pltpu.SemaphoreType.DMA for _ in range(3 * _NSLOT)]
    ),
    compiler_params=pltpu.CompilerParams(use_tc_tiling_on_sc=False),
)
def _gather_kernel(ids_hbm, table_hbm, out_hbm, *scr):
    idx = scr[0:_NSLOT]
    rows = scr[_NSLOT : 2 * _NSLOT]
    isem = scr[2 * _NSLOT : 3 * _NSLOT]
    gsem = scr[3 * _NSLOT : 4 * _NSLOT]
    osem = scr[4 * _NSLOT : 5 * _NSLOT]

    wid = lax.axis_index("s") * _NC + lax.axis_index("c")
    base = wid * _PER_W

    for s in range(_NSLOT):
        pltpu.async_copy(ids_hbm.at[pl.ds(base + s * _C, _C)], idx[s], isem[s])

    def body(t, carry):
        off = base + t * (_NSLOT * _C)
        gathers = []
        for s in range(_NSLOT):
            o = off + s * _C
            pltpu.make_async_copy(ids_hbm.at[pl.ds(o, _C)], idx[s], isem[s]).wait()

            @pl.when(t > 0)
            def _(s=s, o=o):
                pltpu.make_async_copy(
                    rows[s], out_hbm.at[pl.ds(o - _NSLOT * _C, _C)], osem[s]
                ).wait()

            gathers.append(pltpu.async_copy(table_hbm.at[idx[s]], rows[s], gsem[s]))

        for s in range(_NSLOT):
            o = off + s * _C
            gathers[s].wait()
            pltpu.async_copy(rows[s], out_hbm.at[pl.ds(o, _C)], osem[s])

            @pl.when(t < _ITERS - 1)
            def _(s=s, o=o):
                pltpu.async_copy(
                    ids_hbm.at[pl.ds(o + _NSLOT * _C, _C)], idx[s], isem[s]
                )

        return carry

    lax.fori_loop(0, _ITERS, body, 0)

    for s in range(_NSLOT):
        o = base + (_CHUNKS - _NSLOT + s) * _C
        pltpu.make_async_copy(rows[s], out_hbm.at[pl.ds(o, _C)], osem[s]).wait()


_BW = _BATCH // _NW
_G = _BW // 16


@functools.partial(
    pl.kernel,
    mesh=_mesh,
    out_type=jax.ShapeDtypeStruct((_SEQ, _EMBED, _BATCH), jnp.float32),
    scratch_types=(
        [pltpu.VMEM((_BW * _EMBED,), jnp.float32) for _ in range(2)]
        + [pltpu.VMEM((_EMBED, _BW), jnp.float32) for _ in range(2)]
        + [pltpu.SemaphoreType.DMA for _ in range(4)]
    ),
    compiler_params=pltpu.CompilerParams(
        use_tc_tiling_on_sc=True, needs_layout_passes=False
    ),
)
def _transpose_kernel(x_hbm, out_hbm, xv0, xv1, tv0, tv1, is0, is1, os0, os1):
    wid = lax.axis_index("s") * _NC + lax.axis_index("c")
    b0 = wid * _BW
    xv = (xv0, xv1)
    tv = (tv0, tv1)
    isem = (is0, is1)
    osem = (os0, os1)

    lane = lax.iota(jnp.int32, 16)
    rowbase = [(lane + 16 * g) * _EMBED for g in range(_G)]

    def load(s, slot):
        pltpu.async_copy(
            x_hbm.at[pl.ds((s * _BATCH + b0) * _EMBED, _BW * _EMBED)],
            xv[slot],
            isem[slot],
        )

    load(0, 0)
    load(1, 1)
    iters = _SEQ // 2

    def body(t, carry):
        for slot in range(2):
            s = 2 * t + slot
            pltpu.make_async_copy(
                x_hbm.at[pl.ds((s * _BATCH + b0) * _EMBED, _BW * _EMBED)],
                xv[slot],
                isem[slot],
            ).wait()

            @pl.when(t > 0)
            def _(slot=slot, s=s):
                pltpu.make_async_copy(
                    tv[slot], out_hbm.at[s - 2, :, pl.ds(b0, _BW)], osem[slot]
                ).wait()

            for d in range(_EMBED):
                for g in range(_G):
                    tv[slot][d, pl.ds(16 * g, 16)] = plsc.load_gather(
                        xv[slot], [rowbase[g] + d]
                    )

            pltpu.async_copy(tv[slot], out_hbm.at[s, :, pl.ds(b0, _BW)], osem[slot])

            @pl.when(t < iters - 1)
            def _(slot=slot, s=s):
                load(s + 2, slot)

        return carry

    lax.fori_loop(0, iters, body, 0)

    for slot in range(2):
        s = _SEQ - 2 + slot
        pltpu.make_async_copy(
            tv[slot], out_hbm.at[s, :, pl.ds(b0, _BW)], osem[slot]
        ).wait()


def kernel(input_ids, table):
    ids = input_ids.T.reshape(-1).astype(jnp.int32)
    rows = _gather_kernel(ids, table)
    out_t = _transpose_kernel(rows.reshape(-1))
    return out_t.transpose(2, 0, 1)


# speedup vs baseline: 1.8147x; 1.8147x over previous
import functools

import jax
import jax.numpy as jnp
from jax import lax
from jax.experimental import pallas as pl
from jax.experimental.pallas import tpu as pltpu
from jax.experimental.pallas import tpu_sc as plsc

_BATCH = 4096
_SEQ = 200
_EMBED = 64
_B = _BATCH * _SEQ

_info = plsc.get_sparse_core_info()
_NC, _NS = _info.num_cores, _info.num_subcores
_NW = _NC * _NS
_PER_W = _B // _NW
_NSLOT = 4
_C = 400
_CHUNKS = _PER_W // _C
_ITERS = _CHUNKS // _NSLOT

_mesh = plsc.VectorSubcoreMesh(core_axis_name="c", subcore_axis_name="s")


@functools.partial(
    pl.kernel,
    mesh=_mesh,
    out_type=jax.ShapeDtypeStruct((_B, _EMBED), jnp.float32),
    scratch_types=(
        [pltpu.VMEM((_C,), jnp.int32) for _ in range(_NSLOT)]
        + [pltpu.VMEM((_C, _EMBED), jnp.float32) for _ in range(_NSLOT)]
        + [pltpu.SemaphoreType.DMA for _ in range(3 * _NSLOT)]
    ),
    compiler_params=pltpu.CompilerParams(use_tc_tiling_on_sc=False),
)
def _gather_kernel(ids_hbm, table_hbm, out_hbm, *scr):
    idx = scr[0:_NSLOT]
    rows = scr[_NSLOT : 2 * _NSLOT]
    isem = scr[2 * _NSLOT : 3 * _NSLOT]
    gsem = scr[3 * _NSLOT : 4 * _NSLOT]
    osem = scr[4 * _NSLOT : 5 * _NSLOT]

    wid = lax.axis_index("s") * _NC + lax.axis_index("c")
    base = wid * _PER_W

    for s in range(_NSLOT):
        pltpu.async_copy(ids_hbm.at[pl.ds(base + s * _C, _C)], idx[s], isem[s])

    def body(t, carry):
        off = base + t * (_NSLOT * _C)
        gathers = []
        for s in range(_NSLOT):
            o = off + s * _C
            pltpu.make_async_copy(ids_hbm.at[pl.ds(o, _C)], idx[s], isem[s]).wait()

            @pl.when(t > 0)
            def _(s=s, o=o):
                pltpu.make_async_copy(
                    rows[s], out_hbm.at[pl.ds(o - _NSLOT * _C, _C)], osem[s]
                ).wait()

            gathers.append(pltpu.async_copy(table_hbm.at[idx[s]], rows[s], gsem[s]))

        for s in range(_NSLOT):
            o = off + s * _C
            gathers[s].wait()
            pltpu.async_copy(rows[s], out_hbm.at[pl.ds(o, _C)], osem[s])

            @pl.when(t < _ITERS - 1)
            def _(s=s, o=o):
                pltpu.async_copy(
                    ids_hbm.at[pl.ds(o + _NSLOT * _C, _C)], idx[s], isem[s]
                )

        return carry

    lax.fori_loop(0, _ITERS, body, 0)

    for s in range(_NSLOT):
        o = base + (_CHUNKS - _NSLOT + s) * _C
        pltpu.make_async_copy(rows[s], out_hbm.at[pl.ds(o, _C)], osem[s]).wait()


def kernel(input_ids, table):
    ids = input_ids.T.reshape(-1).astype(jnp.int32)
    out = _gather_kernel(ids, table)
    return out.reshape(_SEQ, _BATCH, _EMBED).transpose(1, 0, 2)
